# baseline (device time: 7000 ns/iter reference)
import jax
import jax.numpy as jnp
from jax import lax
from jax.experimental import pallas as pl
from jax.experimental.pallas import tpu as pltpu

N_CHUNK = 4


def kernel(x):
    _, m, n2 = x.shape
    n = n2 // 2
    rows = m // N_CHUNK

    def body(
        x_ref, out_ref, x_vmem, send_buf, recv_buf, acc_buf,
        in_sem, send_sems, recv_sems, out_sems,
    ):
        my_x = lax.axis_index("x")
        my_y = lax.axis_index("y")
        my_z = lax.axis_index("z")

        barrier_sem = pltpu.get_barrier_semaphore()
        pl.semaphore_signal(
            barrier_sem, inc=1,
            device_id=(1 - my_x, my_y, my_z),
            device_id_type=pl.DeviceIdType.MESH,
        )

        cp_in = pltpu.make_async_copy(x_ref.at[0], x_vmem, in_sem)
        cp_in.start()
        cp_in.wait()

        def run(xpos):
            send_off = (1 - xpos) * n
            local_off = xpos * n
            send_buf[:, :] = x_vmem[:, pl.ds(send_off, n)].astype(
                jnp.bfloat16
            )
            pl.semaphore_wait(barrier_sem, 1)
            rdmas = []
            for c in range(N_CHUNK):
                r0 = c * rows
                rdma = pltpu.make_async_remote_copy(
                    src_ref=send_buf.at[pl.ds(r0, rows), :],
                    dst_ref=recv_buf.at[pl.ds(r0, rows), :],
                    send_sem=send_sems.at[c],
                    recv_sem=recv_sems.at[c],
                    device_id=(1 - xpos, my_y, my_z),
                    device_id_type=pl.DeviceIdType.MESH,
                )
                rdma.start()
                rdmas.append(rdma)
            acc_buf[:, :] = x_vmem[:, pl.ds(local_off, n)].astype(
                jnp.bfloat16
            )
            copies = []
            for c, rdma in enumerate(rdmas):
                r0 = c * rows
                rdma.wait_recv()
                acc_buf[pl.ds(r0, rows), :] = (
                    acc_buf[pl.ds(r0, rows), :]
                    + recv_buf[pl.ds(r0, rows), :]
                )
                cp = pltpu.make_async_copy(
                    acc_buf.at[pl.ds(r0, rows), :],
                    out_ref.at[pl.ds(r0, rows), :],
                    out_sems.at[c],
                )
                cp.start()
                copies.append(cp)
            for cp in copies:
                cp.wait()
            for rdma in rdmas:
                rdma.wait_send()

        @pl.when(my_x == 0)
        def _():
            run(0)

        @pl.when(my_x == 1)
        def _():
            run(1)

    return pl.pallas_call(
        body,
        out_shape=jax.ShapeDtypeStruct((m, n), jnp.bfloat16),
        in_specs=[pl.BlockSpec(memory_space=pl.ANY)],
        out_specs=pl.BlockSpec(memory_space=pl.ANY),
        scratch_shapes=[
            pltpu.VMEM((m, n2), jnp.float32),
            pltpu.VMEM((m, n), jnp.bfloat16),
            pltpu.VMEM((m, n), jnp.bfloat16),
            pltpu.VMEM((m, n), jnp.bfloat16),
            pltpu.SemaphoreType.DMA,
            pltpu.SemaphoreType.DMA((N_CHUNK,)),
            pltpu.SemaphoreType.DMA((N_CHUNK,)),
            pltpu.SemaphoreType.DMA((N_CHUNK,)),
        ],
        compiler_params=pltpu.CompilerParams(collective_id=0),
    )(x)


# device time: 6974 ns/iter; 1.0037x vs baseline; 1.0037x over previous
import jax
import jax.numpy as jnp
from jax import lax
from jax.experimental import pallas as pl
from jax.experimental.pallas import tpu as pltpu

N_CHUNK = 4


def kernel(x):
    _, m, n2 = x.shape
    n = n2 // 2
    rows = m // N_CHUNK

    def body(
        x_ref, out_ref, x_vmem, send_buf, recv_buf, acc_buf,
        in_sem, send_sems, recv_sems, out_sems,
    ):
        my_x = lax.axis_index("x")
        my_y = lax.axis_index("y")
        my_z = lax.axis_index("z")

        barrier_sem = pltpu.get_barrier_semaphore()
        pl.semaphore_signal(
            barrier_sem, inc=1,
            device_id=(1 - my_x, my_y, my_z),
            device_id_type=pl.DeviceIdType.MESH,
        )

        cp_in = pltpu.make_async_copy(x_ref.at[0], x_vmem, in_sem)
        cp_in.start()
        cp_in.wait()

        def run(xpos):
            send_off = (1 - xpos) * n
            local_off = xpos * n
            send_buf[:, :] = x_vmem[:, pl.ds(send_off, n)].astype(
                jnp.bfloat16
            )
            pl.semaphore_wait(barrier_sem, 1)
            rdmas = []
            for c in range(N_CHUNK):
                r0 = c * rows
                rdma = pltpu.make_async_remote_copy(
                    src_ref=send_buf.at[pl.ds(r0, rows), :],
                    dst_ref=recv_buf.at[pl.ds(r0, rows), :],
                    send_sem=send_sems.at[c],
                    recv_sem=recv_sems.at[c],
                    device_id=(1 - xpos, my_y, my_z),
                    device_id_type=pl.DeviceIdType.MESH,
                )
                rdma.start()
                rdmas.append(rdma)
            acc_buf[:, :] = x_vmem[:, pl.ds(local_off, n)].astype(
                jnp.bfloat16
            )
            copies = []
            for c, rdma in enumerate(rdmas):
                r0 = c * rows
                rdma.wait_recv()
                acc_buf[pl.ds(r0, rows), :] = (
                    acc_buf[pl.ds(r0, rows), :]
                    + recv_buf[pl.ds(r0, rows), :]
                )
                cp = pltpu.make_async_copy(
                    acc_buf.at[pl.ds(r0, rows), :],
                    out_ref.at[pl.ds(r0, rows), :],
                    out_sems.at[c],
                )
                cp.start()
                copies.append(cp)
            for cp in copies:
                cp.wait()
            for rdma in rdmas:
                rdma.wait_send()

        @pl.when(my_x == 0)
        def _():
            run(0)

        @pl.when(my_x == 1)
        def _():
            run(1)

    return pl.pallas_call(
        body,
        out_shape=jax.ShapeDtypeStruct((m, n), jnp.bfloat16),
        in_specs=[pl.BlockSpec(memory_space=pltpu.MemorySpace.HBM)],
        out_specs=pl.BlockSpec(memory_space=pltpu.MemorySpace.HBM),
        scratch_shapes=[
            pltpu.VMEM((m, n2), jnp.float32),
            pltpu.VMEM((m, n), jnp.bfloat16),
            pltpu.VMEM((m, n), jnp.bfloat16),
            pltpu.VMEM((m, n), jnp.bfloat16),
            pltpu.SemaphoreType.DMA,
            pltpu.SemaphoreType.DMA((N_CHUNK,)),
            pltpu.SemaphoreType.DMA((N_CHUNK,)),
            pltpu.SemaphoreType.DMA((N_CHUNK,)),
        ],
        compiler_params=pltpu.CompilerParams(collective_id=0),
    )(x)


# device time: 6932 ns/iter; 1.0098x vs baseline; 1.0061x over previous
import jax
import jax.numpy as jnp
from jax import lax
from jax.experimental import pallas as pl
from jax.experimental.pallas import tpu as pltpu

N_CHUNK = 4


def kernel(x):
    _, m, n2 = x.shape
    n = n2 // 2
    rows = m // N_CHUNK

    def body(
        x_ref, out_ref, x_vmem, send_buf, recv_buf, acc_buf,
        in_sem, send_sems, recv_sems, out_sems,
    ):
        my_x = lax.axis_index("x")
        my_y = lax.axis_index("y")
        my_z = lax.axis_index("z")

        barrier_sem = pltpu.get_barrier_semaphore()
        pl.semaphore_signal(
            barrier_sem, inc=1,
            device_id=(1 - my_x, my_y, my_z),
            device_id_type=pl.DeviceIdType.MESH,
        )

        cp_in = pltpu.make_async_copy(x_ref.at[0], x_vmem, in_sem)
        cp_in.start()
        cp_in.wait()

        def run(xpos):
            send_off = (1 - xpos) * n
            local_off = xpos * n
            send_buf[:, :] = x_vmem[:, pl.ds(send_off, n)].astype(
                jnp.bfloat16
            )
            pl.semaphore_wait(barrier_sem, 1)
            rdmas = []
            for c in range(N_CHUNK):
                r0 = c * rows
                rdma = pltpu.make_async_remote_copy(
                    src_ref=send_buf.at[pl.ds(r0, rows), :],
                    dst_ref=recv_buf.at[pl.ds(r0, rows), :],
                    send_sem=send_sems.at[c],
                    recv_sem=recv_sems.at[c],
                    device_id=(1 - xpos, my_y, my_z),
                    device_id_type=pl.DeviceIdType.MESH,
                )
                rdma.start()
                rdmas.append(rdma)
            acc_buf[:, :] = x_vmem[:, pl.ds(local_off, n)].astype(
                jnp.bfloat16
            )
            copies = []
            for c, rdma in enumerate(rdmas):
                r0 = c * rows
                rdma.wait_recv()
                acc_buf[pl.ds(r0, rows), :] = (
                    acc_buf[pl.ds(r0, rows), :]
                    + recv_buf[pl.ds(r0, rows), :]
                )
                cp = pltpu.make_async_copy(
                    acc_buf.at[pl.ds(r0, rows), :],
                    out_ref.at[pl.ds(r0, rows), :],
                    out_sems.at[c],
                )
                cp.start()
                copies.append(cp)
            for cp in copies:
                cp.wait()
            for rdma in rdmas:
                rdma.wait_send()

        @pl.when(my_x == 0)
        def _():
            run(0)

        @pl.when(my_x == 1)
        def _():
            run(1)

    return pl.pallas_call(
        body,
        out_shape=jax.ShapeDtypeStruct((m, n), jnp.bfloat16),
        in_specs=[pl.BlockSpec(memory_space=pltpu.MemorySpace.HBM)],
        out_specs=pl.BlockSpec(memory_space=pltpu.MemorySpace.HBM),
        scratch_shapes=[
            pltpu.VMEM((m, n2), jnp.float32),
            pltpu.VMEM((m, n), jnp.bfloat16),
            pltpu.VMEM((m, n), jnp.bfloat16),
            pltpu.VMEM((m, n), jnp.bfloat16),
            pltpu.SemaphoreType.DMA,
            pltpu.SemaphoreType.DMA((N_CHUNK,)),
            pltpu.SemaphoreType.DMA((N_CHUNK,)),
            pltpu.SemaphoreType.DMA((N_CHUNK,)),
        ],
        compiler_params=pltpu.CompilerParams(collective_id=0),
    )(pltpu.with_memory_space_constraint(x, pltpu.MemorySpace.HBM))


# device time: 6744 ns/iter; 1.0380x vs baseline; 1.0279x over previous
import jax
import jax.numpy as jnp
from jax import lax
from jax.experimental import pallas as pl
from jax.experimental.pallas import tpu as pltpu

N_CHUNK = 4


def kernel(x):
    _, m, n2 = x.shape
    n = n2 // 2
    rows = m // N_CHUNK

    def body(
        x_ref, out_ref, xs_vmem, xl_vmem, send_buf, recv_buf, acc_buf,
        in_sems, local_sem, send_sems, recv_sems, out_sems,
    ):
        my_x = lax.axis_index("x")
        my_y = lax.axis_index("y")
        my_z = lax.axis_index("z")

        barrier_sem = pltpu.get_barrier_semaphore()
        pl.semaphore_signal(
            barrier_sem, inc=1,
            device_id=(1 - my_x, my_y, my_z),
            device_id_type=pl.DeviceIdType.MESH,
        )

        def run(xpos):
            send_off = (1 - xpos) * n
            local_off = xpos * n

            pulls = []
            for c in range(N_CHUNK):
                r0 = c * rows
                cp = pltpu.make_async_copy(
                    x_ref.at[0, pl.ds(r0, rows), pl.ds(send_off, n)],
                    xs_vmem.at[pl.ds(r0, rows), :],
                    in_sems.at[c],
                )
                cp.start()
                pulls.append(cp)
            cp_local = pltpu.make_async_copy(
                x_ref.at[0, :, pl.ds(local_off, n)], xl_vmem, local_sem
            )
            cp_local.start()

            rdmas = []
            for c in range(N_CHUNK):
                r0 = c * rows
                pulls[c].wait()
                send_buf[pl.ds(r0, rows), :] = xs_vmem[
                    pl.ds(r0, rows), :
                ].astype(jnp.bfloat16)
                if c == 0:
                    pl.semaphore_wait(barrier_sem, 1)
                rdma = pltpu.make_async_remote_copy(
                    src_ref=send_buf.at[pl.ds(r0, rows), :],
                    dst_ref=recv_buf.at[pl.ds(r0, rows), :],
                    send_sem=send_sems.at[c],
                    recv_sem=recv_sems.at[c],
                    device_id=(1 - xpos, my_y, my_z),
                    device_id_type=pl.DeviceIdType.MESH,
                )
                rdma.start()
                rdmas.append(rdma)

            cp_local.wait()
            acc_buf[:, :] = xl_vmem[:, :].astype(jnp.bfloat16)

            copies = []
            for c, rdma in enumerate(rdmas):
                r0 = c * rows
                rdma.wait_recv()
                acc_buf[pl.ds(r0, rows), :] = (
                    acc_buf[pl.ds(r0, rows), :]
                    + recv_buf[pl.ds(r0, rows), :]
                )
                cp = pltpu.make_async_copy(
                    acc_buf.at[pl.ds(r0, rows), :],
                    out_ref.at[pl.ds(r0, rows), :],
                    out_sems.at[c],
                )
                cp.start()
                copies.append(cp)
            for cp in copies:
                cp.wait()
            for rdma in rdmas:
                rdma.wait_send()

        @pl.when(my_x == 0)
        def _():
            run(0)

        @pl.when(my_x == 1)
        def _():
            run(1)

    return pl.pallas_call(
        body,
        out_shape=jax.ShapeDtypeStruct((m, n), jnp.bfloat16),
        in_specs=[pl.BlockSpec(memory_space=pltpu.MemorySpace.HBM)],
        out_specs=pl.BlockSpec(memory_space=pltpu.MemorySpace.HBM),
        scratch_shapes=[
            pltpu.VMEM((m, n), jnp.float32),
            pltpu.VMEM((m, n), jnp.float32),
            pltpu.VMEM((m, n), jnp.bfloat16),
            pltpu.VMEM((m, n), jnp.bfloat16),
            pltpu.VMEM((m, n), jnp.bfloat16),
            pltpu.SemaphoreType.DMA((N_CHUNK,)),
            pltpu.SemaphoreType.DMA,
            pltpu.SemaphoreType.DMA((N_CHUNK,)),
            pltpu.SemaphoreType.DMA((N_CHUNK,)),
            pltpu.SemaphoreType.DMA((N_CHUNK,)),
        ],
        compiler_params=pltpu.CompilerParams(collective_id=0),
    )(pltpu.with_memory_space_constraint(x, pltpu.MemorySpace.HBM))


# device time: 6665 ns/iter; 1.0503x vs baseline; 1.0119x over previous
import jax
import jax.numpy as jnp
from jax import lax
from jax.experimental import pallas as pl
from jax.experimental.pallas import tpu as pltpu

N_CHUNK = 4


def kernel(x):
    _, m, n2 = x.shape
    n = n2 // 2
    rows = m // N_CHUNK

    def body(
        x_ref, out_ref, xs_vmem, xl_vmem, send_buf, recv_buf,
        in_sems, local_sem, send_sems, recv_sems,
    ):
        my_x = lax.axis_index("x")
        my_y = lax.axis_index("y")
        my_z = lax.axis_index("z")

        barrier_sem = pltpu.get_barrier_semaphore()
        pl.semaphore_signal(
            barrier_sem, inc=1,
            device_id=(1 - my_x, my_y, my_z),
            device_id_type=pl.DeviceIdType.MESH,
        )

        def run(xpos):
            send_off = (1 - xpos) * n
            local_off = xpos * n

            pulls = []
            for c in range(N_CHUNK):
                r0 = c * rows
                cp = pltpu.make_async_copy(
                    x_ref.at[0, pl.ds(r0, rows), pl.ds(send_off, n)],
                    xs_vmem.at[pl.ds(r0, rows), :],
                    in_sems.at[c],
                )
                cp.start()
                pulls.append(cp)
            cp_local = pltpu.make_async_copy(
                x_ref.at[0, :, pl.ds(local_off, n)], xl_vmem, local_sem
            )
            cp_local.start()

            rdmas = []
            for c in range(N_CHUNK):
                r0 = c * rows
                pulls[c].wait()
                send_buf[pl.ds(r0, rows), :] = xs_vmem[
                    pl.ds(r0, rows), :
                ].astype(jnp.bfloat16)
                if c == 0:
                    pl.semaphore_wait(barrier_sem, 1)
                rdma = pltpu.make_async_remote_copy(
                    src_ref=send_buf.at[pl.ds(r0, rows), :],
                    dst_ref=recv_buf.at[pl.ds(r0, rows), :],
                    send_sem=send_sems.at[c],
                    recv_sem=recv_sems.at[c],
                    device_id=(1 - xpos, my_y, my_z),
                    device_id_type=pl.DeviceIdType.MESH,
                )
                rdma.start()
                rdmas.append(rdma)

            cp_local.wait()
            out_ref[:, :] = xl_vmem[:, :].astype(jnp.bfloat16)

            for c, rdma in enumerate(rdmas):
                r0 = c * rows
                rdma.wait_recv()
                out_ref[pl.ds(r0, rows), :] = (
                    out_ref[pl.ds(r0, rows), :]
                    + recv_buf[pl.ds(r0, rows), :]
                )
            for rdma in rdmas:
                rdma.wait_send()

        @pl.when(my_x == 0)
        def _():
            run(0)

        @pl.when(my_x == 1)
        def _():
            run(1)

    return pl.pallas_call(
        body,
        out_shape=jax.ShapeDtypeStruct((m, n), jnp.bfloat16),
        in_specs=[pl.BlockSpec(memory_space=pltpu.MemorySpace.HBM)],
        out_specs=pl.BlockSpec(memory_space=pltpu.VMEM),
        scratch_shapes=[
            pltpu.VMEM((m, n), jnp.float32),
            pltpu.VMEM((m, n), jnp.float32),
            pltpu.VMEM((m, n), jnp.bfloat16),
            pltpu.VMEM((m, n), jnp.bfloat16),
            pltpu.SemaphoreType.DMA((N_CHUNK,)),
            pltpu.SemaphoreType.DMA,
            pltpu.SemaphoreType.DMA((N_CHUNK,)),
            pltpu.SemaphoreType.DMA((N_CHUNK,)),
        ],
        compiler_params=pltpu.CompilerParams(collective_id=0),
    )(pltpu.with_memory_space_constraint(x, pltpu.MemorySpace.HBM))
